# baseline (speedup 1.0000x reference)
import jax
import jax.numpy as jnp
from jax import lax
from jax.experimental import pallas as pl
from jax.experimental.pallas import tpu as pltpu
from jax.experimental.pallas import tpu_sc as plsc

BATCH = 16384
FACTORS = 64
NC = 2
NS = 16
NW = NC * NS
LANES = 16
HITS_PER_TEC = BATCH // NW
GRANULES = HITS_PER_TEC // LANES
NSLOTS = 8
SLAB_W = 128
B_PER_W = BATCH // NW
VECS = B_PER_W // LANES


def _gather_body(tbl, rank2, cu2, pos2, new2, pfb2, pfa2, out1d,
                 st_rank, st_cu, st_pos, st_new, st_pfb, st_pfa,
                 ring, outrow, slabsem, outsem):
    row = lax.axis_index("s") * NC + lax.axis_index("c")
    pltpu.sync_copy(rank2.at[row], st_rank)
    pltpu.sync_copy(cu2.at[row], st_cu)
    pltpu.sync_copy(pos2.at[row], st_pos)
    pltpu.sync_copy(new2.at[row], st_new)
    pltpu.sync_copy(pfb2.at[row], st_pfb)
    pltpu.sync_copy(pfa2.at[row], st_pfa)

    pav = st_pfa[pl.ds(0, LANES)]
    for k in range(NSLOTS - 1):
        tuk = pav[k]

        @pl.when(tuk >= 0)
        def _():
            pltpu.async_copy(
                tbl.at[:, pl.ds(pl.multiple_of(tuk * SLAB_W, SLAB_W), SLAB_W)],
                ring.at[pl.ds(k * FACTORS, FACTORS)], slabsem)

    def granule(g, skip_wait_lanes):
        base_h = g * LANES
        rk = st_rank[pl.ds(base_h, LANES)]
        cu = st_cu[pl.ds(base_h, LANES)]
        po = st_pos[pl.ds(base_h, LANES)]
        nw = st_new[pl.ds(base_h, LANES)]
        pb = st_pfb[pl.ds(base_h, LANES)]
        for k in range(LANES):
            rkk = rk[k]
            tub = pb[k]

            @pl.when(tub >= 0)
            def _():
                dslot = (rkk + NSLOTS - 1) & (NSLOTS - 1)
                pltpu.async_copy(
                    tbl.at[:, pl.ds(pl.multiple_of(tub * SLAB_W, SLAB_W),
                                    SLAB_W)],
                    ring.at[pl.ds(pl.multiple_of(dslot * FACTORS, FACTORS),
                                  FACTORS)],
                    slabsem)

            @pl.when(nw[k] > 0)
            def _():
                pltpu.make_async_copy(
                    tbl.at[:, pl.ds(0, SLAB_W)],
                    ring.at[pl.ds(0, FACTORS)], slabsem).wait()

            if k not in skip_wait_lanes:
                pltpu.make_async_copy(
                    tbl.at[0, pl.ds(0, FACTORS)],
                    outrow.at[k & (NSLOTS - 1)], outsem).wait()

            base = (rkk & (NSLOTS - 1)) * FACTORS
            cuk = jnp.full((LANES,), cu[k], dtype=jnp.int32)
            for cc in range(FACTORS // LANES):
                idx0 = (jnp.full((LANES,), base + cc * LANES, dtype=jnp.int32)
                        + lax.iota(jnp.int32, LANES))
                gv = plsc.load_gather(ring, [idx0, cuk])
                outrow[k & (NSLOTS - 1), pl.ds(cc * LANES, LANES)] = gv
            pltpu.async_copy(
                outrow.at[k & (NSLOTS - 1)],
                out1d.at[pl.ds(pl.multiple_of(po[k] * FACTORS, NSLOTS),
                               FACTORS)],
                outsem)
        return 0

    granule(0, set(range(NSLOTS)))
    lax.fori_loop(1, GRANULES, lambda g, c: granule(g, set()), 0)

    for k in range(NSLOTS):
        pltpu.make_async_copy(
            tbl.at[0, pl.ds(0, FACTORS)], outrow.at[k], outsem).wait()


def _dot_body(u1d_hbm, i1d_hbm, out_hbm, vu, vi, out_v, sem):
    wid = lax.axis_index("s") * NC + lax.axis_index("c")
    n = B_PER_W * FACTORS
    pltpu.sync_copy(u1d_hbm.at[pl.ds(wid * n, n)], vu)
    pltpu.sync_copy(i1d_hbm.at[pl.ds(wid * n, n)], vi)

    def group(g, carry):
        row0 = jnp.full((LANES,), g * LANES * FACTORS, dtype=jnp.int32)
        rows = row0 + lax.iota(jnp.int32, LANES) * FACTORS
        acc = None
        for f in range(FACTORS):
            gu = plsc.load_gather(vu, [rows + f])
            gi = plsc.load_gather(vi, [rows + f])
            acc = gu * gi if acc is None else acc + gu * gi
        out_v[pl.ds(g * LANES, LANES)] = acc
        return carry

    lax.fori_loop(0, VECS, group, 0)
    pltpu.sync_copy(out_v, out_hbm.at[pl.ds(wid * B_PER_W, B_PER_W)])


_MESH = dict(core_axis_name="c", subcore_axis_name="s")


def _gather_call(tblT, prep):
    mesh = plsc.VectorSubcoreMesh(**_MESH)
    return pl.kernel(
        _gather_body,
        mesh=mesh,
        compiler_params=pltpu.CompilerParams(needs_layout_passes=False),
        out_type=jax.ShapeDtypeStruct((BATCH * FACTORS,), jnp.float32),
        scratch_types=[
            pltpu.VMEM((HITS_PER_TEC,), jnp.int32),
            pltpu.VMEM((HITS_PER_TEC,), jnp.int32),
            pltpu.VMEM((HITS_PER_TEC,), jnp.int32),
            pltpu.VMEM((HITS_PER_TEC,), jnp.int32),
            pltpu.VMEM((HITS_PER_TEC,), jnp.int32),
            pltpu.VMEM((LANES,), jnp.int32),
            pltpu.VMEM((NSLOTS * FACTORS, SLAB_W), jnp.float32),
            pltpu.VMEM((NSLOTS, FACTORS), jnp.float32),
            pltpu.SemaphoreType.DMA,
            pltpu.SemaphoreType.DMA,
        ],
    )(tblT, *prep)


@jax.jit
def _pointmf_sc(uT, iT, prep_u, prep_i):
    out_u = _gather_call(uT, prep_u)
    out_i = _gather_call(iT, prep_i)
    mesh = plsc.VectorSubcoreMesh(**_MESH)
    return pl.kernel(
        _dot_body,
        mesh=mesh,
        compiler_params=pltpu.CompilerParams(needs_layout_passes=False),
        out_type=jax.ShapeDtypeStruct((BATCH,), jnp.float32),
        scratch_types=[
            pltpu.VMEM((B_PER_W * FACTORS,), jnp.float32),
            pltpu.VMEM((B_PER_W * FACTORS,), jnp.float32),
            pltpu.VMEM((B_PER_W,), jnp.float32),
            pltpu.SemaphoreType.DMA,
        ],
    )(out_u, out_i)


def _prep(ids):
    pos = lax.iota(jnp.int32, BATCH)
    srt = jnp.sort((ids >> 7) * BATCH + pos)
    pos_s = srt & (BATCH - 1)
    tu_s = srt >> 14
    cu_s = ids[pos_s] & (SLAB_W - 1)

    tu2 = tu_s.reshape(NW, HITS_PER_TEC)
    prev = jnp.concatenate([jnp.full((NW, 1), -1, jnp.int32), tu2[:, :-1]],
                           axis=1)
    is_new = (tu2 != prev).astype(jnp.int32)
    rank = jnp.cumsum(is_new, axis=1) - 1

    segrow = jnp.broadcast_to(jnp.arange(NW)[:, None], (NW, HITS_PER_TEC))
    tor = jnp.full((NW, HITS_PER_TEC), -1, jnp.int32).at[segrow, rank].set(tu2)
    r7 = rank + NSLOTS - 1
    pfb = jnp.where(
        (is_new > 0) & (r7 < HITS_PER_TEC),
        jnp.take_along_axis(tor, jnp.minimum(r7, HITS_PER_TEC - 1), axis=1),
        -1)
    pfa = jnp.concatenate(
        [tor[:, :NSLOTS - 1],
         jnp.full((NW, LANES - (NSLOTS - 1)), -1, jnp.int32)], axis=1)

    return (rank, cu_s.reshape(NW, HITS_PER_TEC),
            pos_s.reshape(NW, HITS_PER_TEC), is_new, pfb, pfa)


def kernel(user, item, context, embed_user_w, embed_item_w):
    del context
    prep_u = _prep(user.astype(jnp.int32))
    prep_i = _prep(item.astype(jnp.int32))
    return _pointmf_sc(embed_user_w.T, embed_item_w.T, prep_u, prep_i)


# speedup vs baseline: 1.0609x; 1.0609x over previous
import jax
import jax.numpy as jnp
from jax import lax
from jax.experimental import pallas as pl
from jax.experimental.pallas import tpu as pltpu
from jax.experimental.pallas import tpu_sc as plsc

BATCH = 16384
FACTORS = 64
NC = 2
NS = 16
NW = NC * NS
LANES = 16
HITS_PER_TEC = BATCH // NS
GRANULES = HITS_PER_TEC // LANES
NSLOTS = 8
SLAB_W = 128
B_PER_W = BATCH // NW
VECS = B_PER_W // LANES
NSEG = 2 * NS


def _gather_run(tbl, rank2, cu2, pos2, new2, pfb2, pfa2, out1d, row,
                st_rank, st_cu, st_pos, st_new, st_pfb, st_pfa,
                ring, outrow, slabsem, outsem):
    pltpu.sync_copy(rank2.at[row], st_rank)
    pltpu.sync_copy(cu2.at[row], st_cu)
    pltpu.sync_copy(pos2.at[row], st_pos)
    pltpu.sync_copy(new2.at[row], st_new)
    pltpu.sync_copy(pfb2.at[row], st_pfb)
    pltpu.sync_copy(pfa2.at[row], st_pfa)

    pav = st_pfa[pl.ds(0, LANES)]
    for k in range(NSLOTS - 1):
        tuk = pav[k]

        @pl.when(tuk >= 0)
        def _():
            pltpu.async_copy(
                tbl.at[:, pl.ds(pl.multiple_of(tuk * SLAB_W, SLAB_W), SLAB_W)],
                ring.at[pl.ds(k * FACTORS, FACTORS)], slabsem)

    def granule(g, skip_wait_lanes):
        base_h = g * LANES
        rk = st_rank[pl.ds(base_h, LANES)]
        cu = st_cu[pl.ds(base_h, LANES)]
        po = st_pos[pl.ds(base_h, LANES)]
        nw = st_new[pl.ds(base_h, LANES)]
        pb = st_pfb[pl.ds(base_h, LANES)]
        for k in range(LANES):
            rkk = rk[k]
            tub = pb[k]

            @pl.when(tub >= 0)
            def _():
                dslot = (rkk + NSLOTS - 1) & (NSLOTS - 1)
                pltpu.async_copy(
                    tbl.at[:, pl.ds(pl.multiple_of(tub * SLAB_W, SLAB_W),
                                    SLAB_W)],
                    ring.at[pl.ds(pl.multiple_of(dslot * FACTORS, FACTORS),
                                  FACTORS)],
                    slabsem)

            @pl.when(nw[k] > 0)
            def _():
                pltpu.make_async_copy(
                    tbl.at[:, pl.ds(0, SLAB_W)],
                    ring.at[pl.ds(0, FACTORS)], slabsem).wait()

            if k not in skip_wait_lanes:
                pltpu.make_async_copy(
                    tbl.at[0, pl.ds(0, FACTORS)],
                    outrow.at[k & (NSLOTS - 1)], outsem).wait()

            base = (rkk & (NSLOTS - 1)) * FACTORS
            cuk = jnp.full((LANES,), cu[k], dtype=jnp.int32)
            for cc in range(FACTORS // LANES):
                idx0 = (jnp.full((LANES,), base + cc * LANES, dtype=jnp.int32)
                        + lax.iota(jnp.int32, LANES))
                gv = plsc.load_gather(ring, [idx0, cuk])
                outrow[k & (NSLOTS - 1), pl.ds(cc * LANES, LANES)] = gv
            pltpu.async_copy(
                outrow.at[k & (NSLOTS - 1)],
                out1d.at[pl.ds(pl.multiple_of(po[k] * FACTORS, NSLOTS),
                               FACTORS)],
                outsem)
        return 0

    granule(0, set(range(NSLOTS)))
    lax.fori_loop(1, GRANULES, lambda g, c: granule(g, set()), 0)

    for k in range(NSLOTS):
        pltpu.make_async_copy(
            tbl.at[0, pl.ds(0, FACTORS)], outrow.at[k], outsem).wait()


def _gather_body(uT, iT, rank2, cu2, pos2, new2, pfb2, pfa2,
                 out_u, out_i,
                 st_rank, st_cu, st_pos, st_new, st_pfb, st_pfa,
                 ring, outrow, slabsem, outsem):
    c = lax.axis_index("c")
    s = lax.axis_index("s")
    row = c * NS + s

    @pl.when(c == 0)
    def _():
        _gather_run(uT, rank2, cu2, pos2, new2, pfb2, pfa2, out_u, row,
                    st_rank, st_cu, st_pos, st_new, st_pfb, st_pfa,
                    ring, outrow, slabsem, outsem)

    @pl.when(c == 1)
    def _():
        _gather_run(iT, rank2, cu2, pos2, new2, pfb2, pfa2, out_i, row,
                    st_rank, st_cu, st_pos, st_new, st_pfb, st_pfa,
                    ring, outrow, slabsem, outsem)


def _dot_body(u1d_hbm, i1d_hbm, out_hbm, vu, vi, out_v, sem):
    wid = lax.axis_index("s") * NC + lax.axis_index("c")
    n = B_PER_W * FACTORS
    pltpu.sync_copy(u1d_hbm.at[pl.ds(wid * n, n)], vu)
    pltpu.sync_copy(i1d_hbm.at[pl.ds(wid * n, n)], vi)

    def group(g, carry):
        row0 = jnp.full((LANES,), g * LANES * FACTORS, dtype=jnp.int32)
        rows = row0 + lax.iota(jnp.int32, LANES) * FACTORS
        acc = None
        for f in range(FACTORS):
            gu = plsc.load_gather(vu, [rows + f])
            gi = plsc.load_gather(vi, [rows + f])
            acc = gu * gi if acc is None else acc + gu * gi
        out_v[pl.ds(g * LANES, LANES)] = acc
        return carry

    lax.fori_loop(0, VECS, group, 0)
    pltpu.sync_copy(out_v, out_hbm.at[pl.ds(wid * B_PER_W, B_PER_W)])


@jax.jit
def _pointmf_sc(uT, iT, prep):
    mesh = plsc.VectorSubcoreMesh(core_axis_name="c", subcore_axis_name="s")
    out_u, out_i = pl.kernel(
        _gather_body,
        mesh=mesh,
        compiler_params=pltpu.CompilerParams(needs_layout_passes=False),
        out_type=(jax.ShapeDtypeStruct((BATCH * FACTORS,), jnp.float32),
                  jax.ShapeDtypeStruct((BATCH * FACTORS,), jnp.float32)),
        scratch_types=[
            pltpu.VMEM((HITS_PER_TEC,), jnp.int32),
            pltpu.VMEM((HITS_PER_TEC,), jnp.int32),
            pltpu.VMEM((HITS_PER_TEC,), jnp.int32),
            pltpu.VMEM((HITS_PER_TEC,), jnp.int32),
            pltpu.VMEM((HITS_PER_TEC,), jnp.int32),
            pltpu.VMEM((LANES,), jnp.int32),
            pltpu.VMEM((NSLOTS * FACTORS, SLAB_W), jnp.float32),
            pltpu.VMEM((NSLOTS, FACTORS), jnp.float32),
            pltpu.SemaphoreType.DMA,
            pltpu.SemaphoreType.DMA,
        ],
    )(uT, iT, *prep)

    return pl.kernel(
        _dot_body,
        mesh=mesh,
        compiler_params=pltpu.CompilerParams(needs_layout_passes=False),
        out_type=jax.ShapeDtypeStruct((BATCH,), jnp.float32),
        scratch_types=[
            pltpu.VMEM((B_PER_W * FACTORS,), jnp.float32),
            pltpu.VMEM((B_PER_W * FACTORS,), jnp.float32),
            pltpu.VMEM((B_PER_W,), jnp.float32),
            pltpu.SemaphoreType.DMA,
        ],
    )(out_u, out_i)


def _prep_both(user, item):
    ids2 = jnp.concatenate([user, item])
    pos2 = lax.iota(jnp.int32, 2 * BATCH)
    table = pos2 >> 14
    key = (table * 8192 + (ids2 >> 7)) * (2 * BATCH) + pos2
    cu = ids2 & (SLAB_W - 1)
    srt, cu_s = lax.sort((key, cu), num_keys=1)
    pos_s2 = srt & (2 * BATCH - 1)
    tu_s = (srt >> 15) & 0x1FFF
    pos_s = pos_s2 & (BATCH - 1)

    tu2 = tu_s.reshape(NSEG, HITS_PER_TEC)
    prev = jnp.concatenate([jnp.full((NSEG, 1), -1, jnp.int32), tu2[:, :-1]],
                           axis=1)
    is_new = (tu2 != prev).astype(jnp.int32)
    rank = jnp.cumsum(is_new, axis=1) - 1

    segrow = jnp.broadcast_to(jnp.arange(NSEG)[:, None], (NSEG, HITS_PER_TEC))
    tor = jnp.full((NSEG, HITS_PER_TEC), -1,
                   jnp.int32).at[segrow, rank].set(tu2)
    r7 = rank + NSLOTS - 1
    pfb = jnp.where(
        (is_new > 0) & (r7 < HITS_PER_TEC),
        jnp.take_along_axis(tor, jnp.minimum(r7, HITS_PER_TEC - 1), axis=1),
        -1)
    pfa = jnp.concatenate(
        [tor[:, :NSLOTS - 1],
         jnp.full((NSEG, LANES - (NSLOTS - 1)), -1, jnp.int32)], axis=1)

    return (rank, cu_s.reshape(NSEG, HITS_PER_TEC),
            pos_s.reshape(NSEG, HITS_PER_TEC), is_new, pfb, pfa)


def kernel(user, item, context, embed_user_w, embed_item_w):
    del context
    prep = _prep_both(user.astype(jnp.int32), item.astype(jnp.int32))
    return _pointmf_sc(embed_user_w.T, embed_item_w.T, prep)
